# baseline (speedup 1.0000x reference)
import jax
import jax.numpy as jnp
from jax.experimental import pallas as pl

NUM_CENTROIDS_K = 1024
COMMIT = 0.25


def _vq_block(x_ref, cb_ref, q_ref, loss_ref, idx_ref):
    pid = pl.program_id(0)
    x = x_ref[0]
    cb = cb_ref[...]
    xsq = jnp.sum(x * x, axis=1, keepdims=True)
    csq = jnp.sum(cb * cb, axis=1)[None, :]
    xc = jax.lax.dot_general(
        x, cb, (((1,), (1,)), ((), ())),
        preferred_element_type=jnp.float32)
    d = xsq - 2.0 * xc + csq
    dmin = jnp.min(d, axis=1, keepdims=True)
    cols = jax.lax.broadcasted_iota(jnp.int32, d.shape, 1)
    idx = jnp.min(jnp.where(d == dmin, cols, NUM_CENTROIDS_K),
                  axis=1).astype(jnp.int32)
    idx_ref[0, pid, :] = idx
    oh = (cols == idx[:, None]).astype(jnp.float32)
    q = jax.lax.dot_general(
        oh, cb, (((1,), (0,)), ((), ())),
        preferred_element_type=jnp.float32)
    diff = q - x
    sq = diff * diff
    loss_ref[0] = sq + COMMIT * sq
    q_ref[0] = x + diff


def kernel(inputs, train, codebook, cluster_counts):
    b, r, dim = inputs.shape
    q_st, loss, idx = pl.pallas_call(
        _vq_block,
        grid=(b,),
        in_specs=[
            pl.BlockSpec((1, r, dim), lambda i: (i, 0, 0)),
            pl.BlockSpec((NUM_CENTROIDS_K, dim), lambda i: (0, 0)),
        ],
        out_specs=[
            pl.BlockSpec((1, r, dim), lambda i: (i, 0, 0)),
            pl.BlockSpec((1, r, dim), lambda i: (i, 0, 0)),
            pl.BlockSpec((1, b, r), lambda i: (0, 0, 0)),
        ],
        out_shape=[
            jax.ShapeDtypeStruct((b, r, dim), jnp.float32),
            jax.ShapeDtypeStruct((b, r, dim), jnp.float32),
            jax.ShapeDtypeStruct((1, b, r), jnp.int32),
        ],
    )(inputs, codebook)
    codebook_values = jax.lax.stop_gradient(codebook[None, ...])
    return q_st, loss, idx, codebook_values, cluster_counts


# speedup vs baseline: 1.0343x; 1.0343x over previous
import jax
import jax.numpy as jnp
from jax.experimental import pallas as pl
from jax.experimental.pallas import tpu as pltpu

NUM_CENTROIDS_K = 1024
COMMIT = 0.25


def _vq_block(x_ref, cb_ref, q_ref, loss_ref, idx_ref, csq_ref):
    pid = pl.program_id(0)
    x = x_ref[0]
    cb = cb_ref[...]

    @pl.when(pid == 0)
    def _():
        csq_ref[...] = jnp.sum(cb * cb, axis=1)[None, :]

    xsq = jnp.sum(x * x, axis=1, keepdims=True)
    xc = jax.lax.dot_general(
        x, cb, (((1,), (1,)), ((), ())),
        preferred_element_type=jnp.float32)
    d = xsq - 2.0 * xc + csq_ref[...]
    dmin = jnp.min(d, axis=1, keepdims=True)
    cols = jax.lax.broadcasted_iota(jnp.int32, d.shape, 1).astype(jnp.float32)
    idx_f = jnp.min(jnp.where(d == dmin, cols, float(NUM_CENTROIDS_K)),
                    axis=1)
    idx_ref[0, pid, :] = idx_f.astype(jnp.int32)
    oh = (cols == idx_f[:, None]).astype(jnp.float32)
    q = jax.lax.dot_general(
        oh, cb, (((1,), (0,)), ((), ())),
        preferred_element_type=jnp.float32)
    diff = q - x
    sq = diff * diff
    loss_ref[0] = sq + COMMIT * sq
    q_ref[0] = x + diff


def kernel(inputs, train, codebook, cluster_counts):
    b, r, dim = inputs.shape
    q_st, loss, idx = pl.pallas_call(
        _vq_block,
        grid=(b,),
        in_specs=[
            pl.BlockSpec((1, r, dim), lambda i: (i, 0, 0)),
            pl.BlockSpec((NUM_CENTROIDS_K, dim), lambda i: (0, 0)),
        ],
        out_specs=[
            pl.BlockSpec((1, r, dim), lambda i: (i, 0, 0)),
            pl.BlockSpec((1, r, dim), lambda i: (i, 0, 0)),
            pl.BlockSpec((1, b, r), lambda i: (0, 0, 0)),
        ],
        out_shape=[
            jax.ShapeDtypeStruct((b, r, dim), jnp.float32),
            jax.ShapeDtypeStruct((b, r, dim), jnp.float32),
            jax.ShapeDtypeStruct((1, b, r), jnp.int32),
        ],
        scratch_shapes=[pltpu.VMEM((1, NUM_CENTROIDS_K), jnp.float32)],
    )(inputs, codebook)
    codebook_values = jax.lax.stop_gradient(codebook[None, ...])
    return q_st, loss, idx, codebook_values, cluster_counts


# speedup vs baseline: 1.2852x; 1.2426x over previous
import jax
import jax.numpy as jnp
from jax.experimental import pallas as pl
from jax.experimental.pallas import tpu as pltpu

NUM_CENTROIDS_K = 1024
COMMIT = 0.25


def _vq_block(xt_ref, cb_ref, q_ref, loss_ref, idx_ref, csq_ref):
    pid = pl.program_id(0)
    x = jnp.swapaxes(xt_ref[0], 0, 1)
    cb = cb_ref[...]

    @pl.when(pid == 0)
    def _():
        csq_ref[...] = jnp.sum(cb * cb, axis=1)[None, :]

    xsq = jnp.sum(x * x, axis=1, keepdims=True)
    xc = jax.lax.dot_general(
        x, cb, (((1,), (1,)), ((), ())),
        preferred_element_type=jnp.float32)
    d = xsq - 2.0 * xc + csq_ref[...]
    dmin = jnp.min(d, axis=1, keepdims=True)
    cols = jax.lax.broadcasted_iota(jnp.int32, d.shape, 1).astype(jnp.float32)
    idx_f = jnp.min(jnp.where(d == dmin, cols, float(NUM_CENTROIDS_K)),
                    axis=1)
    idx_ref[0, pid, :] = idx_f.astype(jnp.int32)
    oh = (cols == idx_f[:, None]).astype(jnp.float32)
    q = jax.lax.dot_general(
        oh, cb, (((1,), (0,)), ((), ())),
        preferred_element_type=jnp.float32)
    diff = q - x
    sq = diff * diff
    loss_ref[0] = jnp.swapaxes(sq + COMMIT * sq, 0, 1)
    q_ref[0] = jnp.swapaxes(x + diff, 0, 1)


def kernel(inputs, train, codebook, cluster_counts):
    b, r, dim = inputs.shape
    xt = jnp.swapaxes(inputs, 1, 2)
    q_st_t, loss_t, idx = pl.pallas_call(
        _vq_block,
        grid=(b,),
        in_specs=[
            pl.BlockSpec((1, dim, r), lambda i: (i, 0, 0)),
            pl.BlockSpec((NUM_CENTROIDS_K, dim), lambda i: (0, 0)),
        ],
        out_specs=[
            pl.BlockSpec((1, dim, r), lambda i: (i, 0, 0)),
            pl.BlockSpec((1, dim, r), lambda i: (i, 0, 0)),
            pl.BlockSpec((1, b, r), lambda i: (0, 0, 0)),
        ],
        out_shape=[
            jax.ShapeDtypeStruct((b, dim, r), jnp.float32),
            jax.ShapeDtypeStruct((b, dim, r), jnp.float32),
            jax.ShapeDtypeStruct((1, b, r), jnp.int32),
        ],
        scratch_shapes=[pltpu.VMEM((1, NUM_CENTROIDS_K), jnp.float32)],
    )(xt, codebook)
    q_st = jnp.swapaxes(q_st_t, 1, 2)
    loss = jnp.swapaxes(loss_t, 1, 2)
    codebook_values = jax.lax.stop_gradient(codebook[None, ...])
    return q_st, loss, idx, codebook_values, cluster_counts
